# Initial kernel scaffold; baseline (speedup 1.0000x reference)
import jax
import jax.numpy as jnp
from jax.experimental import pallas as pl


def kernel(lsm, target):
    raise NotImplementedError("write your pallas kernel here")


# speedup vs baseline: 2.2700x; 2.2700x over previous
import jax
import jax.numpy as jnp
from jax.experimental import pallas as pl
from jax.experimental.pallas import tpu as pltpu

_N_CLASSES = 100000
_SMOOTHING = 0.1
_CONFIDENCE = 1.0 - _SMOOTHING
_EPS = _SMOOTHING / (_N_CLASSES - 1)

_ROWS = 1024
_BLOCK_W = 2048
_NUM_BLOCKS = (_N_CLASSES + _BLOCK_W - 1) // _BLOCK_W


def _loss_kernel(lsm_ref, tgt_ref, out_ref):
    j = pl.program_id(0)
    c0 = j * _BLOCK_W
    blk = lsm_ref[...]
    col = jax.lax.broadcasted_iota(jnp.int32, (_ROWS, _BLOCK_W), 1) + c0
    blk0 = jnp.where(col < _N_CLASSES, blk, 0.0)
    s = jnp.sum(blk0)
    tgt = tgt_ref[...]
    g = jnp.sum(jnp.where(col == tgt, blk, 0.0))
    acc = _EPS * s + (_CONFIDENCE - _EPS) * g

    @pl.when(j == 0)
    def _():
        out_ref[...] = jnp.zeros_like(out_ref)

    out_ref[...] += acc


def kernel(lsm, target):
    tgt = target.astype(jnp.int32).reshape(_ROWS, 1)
    total = pl.pallas_call(
        _loss_kernel,
        grid=(_NUM_BLOCKS,),
        in_specs=[
            pl.BlockSpec((_ROWS, _BLOCK_W), lambda j: (0, j)),
            pl.BlockSpec((_ROWS, 1), lambda j: (0, 0)),
        ],
        out_specs=pl.BlockSpec((1, 1), lambda j: (0, 0)),
        out_shape=jax.ShapeDtypeStruct((1, 1), jnp.float32),
    )(lsm, tgt)
    return -total[0, 0] / _ROWS
